# Initial kernel scaffold; baseline (speedup 1.0000x reference)
import jax
import jax.numpy as jnp
from jax.experimental import pallas as pl


def kernel(inputs, k):
    raise NotImplementedError("write your pallas kernel here")


# speedup vs baseline: 6.0387x; 6.0387x over previous
import functools

import jax
import jax.numpy as jnp
from jax import lax
from jax.experimental import pallas as pl
from jax.experimental.pallas import tpu as pltpu
from jax.experimental.pallas import tpu_sc as plsc

R, C = 128, 32768
KSEL = 512
NW = 32
R_PER = R // NW
NV = C // 16
HSTRIDE = 2048
PASSES = ((21, 11), (10, 11), (0, 10))

@functools.cache
def _get_sc_select():
    mesh = plsc.VectorSubcoreMesh(
        core_axis_name="c", subcore_axis_name="s", num_cores=2, num_subcores=16)
    return pl.kernel(
        _sc_select_body,
        out_type=jax.ShapeDtypeStruct((NW, 16), jnp.int32),
        mesh=mesh,
        scratch_types=[
            pltpu.VMEM((C,), jnp.float32),
            pltpu.VMEM((16 * HSTRIDE,), jnp.int32),
            pltpu.VMEM((HSTRIDE,), jnp.int32),
            pltpu.VMEM((16,), jnp.int32),
        ],
        compiler_params=pltpu.CompilerParams(needs_layout_passes=False),
    )


def _sc_select_body(in_hbm, out_hbm, row_v, hist_v, tot_v, out_v):
    wid = lax.axis_index("s") * 2 + lax.axis_index("c")
    lane = lax.iota(jnp.int32, 16)
    ones = jnp.ones((16,), jnp.int32)
    zeros = jnp.zeros((16,), jnp.int32)
    lane_off = lane * HSTRIDE

    def _zero(i, _):
        hist_v[pl.ds(i * 16, 16)] = zeros
        return 0

    lax.fori_loop(0, 16 * HSTRIDE // 16, _zero, 0)

    def one_pass(shift, nbits, pshift, prefix, k_rem):
        nb = 1 << nbits
        bmask = nb - 1

        def _hist(i, _):
            for u in range(4):
                v = row_v[pl.ds((i * 4 + u) * 16, 16)]
                b = lax.bitcast_convert_type(v, jnp.int32)
                bucket = lax.shift_right_logical(b, shift) & bmask
                idx = lane_off + bucket
                if pshift is None:
                    plsc.addupdate_scatter(hist_v, [idx], ones)
                else:
                    m = lax.shift_right_logical(b, pshift) == prefix
                    plsc.addupdate_scatter(hist_v, [idx], ones, mask=m)
            return 0

        lax.fori_loop(0, NV // 4, _hist, 0)

        def _red(j, _):
            acc = hist_v[pl.ds(j * 16, 16)]
            hist_v[pl.ds(j * 16, 16)] = zeros
            for l in range(1, 16):
                s = l * HSTRIDE + j * 16
                acc = acc + hist_v[pl.ds(s, 16)]
                hist_v[pl.ds(s, 16)] = zeros
            tot_v[pl.ds(j * 16, 16)] = acc
            return 0

        lax.fori_loop(0, nb // 16, _red, 0)

        BIG = jnp.int32(1 << 30)

        def _scan(j, carry):
            cum, found, a_acc = carry
            base = nb - 16 * (j + 1)
            h = tot_v[pl.ds(base, 16)]
            hr = lax.rev(h, (0,))
            cs = lax.cumsum(hr, axis=0) + cum
            hit = cs >= k_rem
            bidx = (base + 15) - lane
            found = jnp.maximum(found, jnp.where(hit, bidx, -1))
            a_acc = jnp.minimum(a_acc, jnp.where(hit, cs - hr, BIG))
            return (cum + jnp.sum(h), found, a_acc)

        _, found, a_acc = lax.fori_loop(
            0, nb // 16, _scan,
            (jnp.int32(0), jnp.full((16,), -1, jnp.int32),
             jnp.full((16,), 1 << 30, jnp.int32)))
        cstar = jnp.max(found)
        a_p = jnp.min(a_acc)
        return cstar, a_p

    def do_row(r, out_acc):
        row = wid * R_PER + r
        pltpu.sync_copy(in_hbm.at[row], row_v)
        k_rem = jnp.int32(KSEL)
        n_above = jnp.int32(0)
        c1, a1 = one_pass(21, 11, None, None, k_rem)
        n_above, k_rem = n_above + a1, k_rem - a1
        c2, a2 = one_pass(10, 11, 21, c1, k_rem)
        n_above, k_rem = n_above + a2, k_rem - a2
        prefix22 = (c1 << 11) | c2
        c3, a3 = one_pass(0, 10, 10, prefix22, k_rem)
        n_above = n_above + a3
        t_bits = (prefix22 << 10) | c3
        out_acc = jnp.where(lane == r, t_bits, out_acc)
        out_acc = jnp.where(lane == R_PER + r, n_above, out_acc)
        return out_acc

    out_acc = jnp.zeros((16,), jnp.int32)
    for r in range(R_PER):
        out_acc = do_row(r, out_acc)
    out_v[...] = out_acc
    pltpu.sync_copy(out_v, out_hbm.at[wid])


_TC_BLK = 2048


def _tc_body(x_ref, t_ref, a_ref, out_ref):
    step = pl.program_id(0)
    x = x_ref[...]
    t = t_ref[...]
    s = jnp.sum(jnp.where(x > t, jnp.log(x), 0.0),
                axis=(0, 1), keepdims=True)

    @pl.when(step == 0)
    def _():
        nsel = jnp.float32(KSEL) - a_ref[...].astype(jnp.float32)
        out_ref[...] = jnp.sum(nsel * jnp.log(t), axis=(0, 1), keepdims=True)

    out_ref[...] += s


def _tc_logsum(x, t, a):
    return pl.pallas_call(
        _tc_body,
        grid=(C // _TC_BLK,),
        in_specs=[
            pl.BlockSpec((R, _TC_BLK), lambda i: (0, i)),
            pl.BlockSpec((R, 1), lambda i: (0, 0)),
            pl.BlockSpec((R, 1), lambda i: (0, 0)),
        ],
        out_specs=pl.BlockSpec((1, 1), lambda i: (0, 0)),
        out_shape=jax.ShapeDtypeStruct((1, 1), jnp.float32),
    )(x, t, a)


def kernel(inputs, k):
    sel = _get_sc_select()(inputs)
    t_bits = sel[:, :R_PER].reshape(R, 1)
    n_above = sel[:, R_PER:2 * R_PER].reshape(R, 1)
    t = lax.bitcast_convert_type(t_bits, jnp.float32)
    total = _tc_logsum(inputs, t, n_above)[0, 0]
    return -total / (jnp.float32(R) * jnp.asarray(k, jnp.float32))


# speedup vs baseline: 19.4636x; 3.2231x over previous
import functools

import jax
import jax.numpy as jnp
from jax import lax
from jax.experimental import pallas as pl
from jax.experimental.pallas import tpu as pltpu
from jax.experimental.pallas import tpu_sc as plsc

R, C = 128, 32768
KSEL = 512
NW = 32
R_PER = R // NW
NV = C // 16
NB1 = 1024
BIG = 1 << 30


@functools.cache
def _get_sc_select():
    mesh = plsc.VectorSubcoreMesh(
        core_axis_name="c", subcore_axis_name="s", num_cores=2, num_subcores=16)
    return pl.kernel(
        _sc_select_body,
        out_type=jax.ShapeDtypeStruct((NW, 16), jnp.int32),
        mesh=mesh,
        scratch_types=[
            pltpu.VMEM((C,), jnp.float32),
            pltpu.VMEM((NB1,), jnp.int32),
            pltpu.VMEM((16,), jnp.int32),
        ],
        compiler_params=pltpu.CompilerParams(needs_layout_passes=False),
    )


def _sc_select_body(in_hbm, out_hbm, row_v, hist_v, out_v):
    wid = lax.axis_index("s") * 2 + lax.axis_index("c")
    lane = lax.iota(jnp.int32, 16)
    zeros16 = jnp.zeros((16,), jnp.int32)

    def _zero(i, _):
        hist_v[pl.ds(i * 16, 16)] = zeros16
        return 0

    lax.fori_loop(0, NB1 // 16, _zero, 0)

    def hist_pass(nvec, shift, bmask, m):
        @plsc.parallel_loop(0, nvec, unroll=4)
        def _(i):
            v = row_v[pl.ds(i * 16, 16)]
            b = lax.bitcast_convert_type(v, jnp.int32)
            bucket = lax.shift_right_logical(b, shift) & bmask
            valid = None if m is None else (i * 16 + lane) < m
            counts, lastm = plsc.scan_count(bucket, valid)
            plsc.addupdate_scatter(hist_v, [bucket], counts, mask=lastm)

    def scan_pass(nb, k_rem):
        def body(j, carry):
            cum, found, a_acc = carry
            base = nb - 16 * (j + 1)
            h = hist_v[pl.ds(base, 16)]
            hist_v[pl.ds(base, 16)] = zeros16
            hr = lax.rev(h, (0,))
            cs = lax.cumsum(hr, axis=0) + cum
            hit = cs >= k_rem
            bidx = (base + 15) - lane
            found = jnp.maximum(found, jnp.where(hit, bidx, -1))
            a_acc = jnp.minimum(a_acc, jnp.where(hit, cs - hr, BIG))
            return (cum + jnp.sum(h), found, a_acc)

        _, found, a_acc = lax.fori_loop(
            0, nb // 16, body,
            (jnp.int32(0), jnp.full((16,), -1, jnp.int32),
             jnp.full((16,), BIG, jnp.int32)))
        return jnp.max(found), jnp.min(a_acc)

    def compact_pass(nvec, shift, bmask, c_sel, m):
        @plsc.parallel_loop(0, nvec, unroll=4, carry=zeros16)
        def off(i, off):
            v = row_v[pl.ds(i * 16, 16)]
            b = lax.bitcast_convert_type(v, jnp.int32)
            key = lax.shift_right_logical(b, shift)
            if bmask is not None:
                key = key & bmask
            keep = key == c_sel
            if m is not None:
                keep = keep & ((i * 16 + lane) < m)
            pos = lax.cumsum(keep.astype(jnp.int32), axis=0)
            plsc.store_scatter(row_v, [off + pos - 1], v, mask=keep)
            return off + plsc.all_reduce_population_count(keep)

        return jnp.max(off)

    def do_row(r, out_acc):
        row = wid * R_PER + r
        pltpu.sync_copy(in_hbm.at[row], row_v)
        hist_pass(NV, 21, NB1 - 1, None)
        c1, a1 = scan_pass(NB1, jnp.int32(KSEL))
        k_rem = jnp.int32(KSEL) - a1
        m1 = compact_pass(NV, 21, None, c1, None)
        nv1 = (m1 + 15) // 16
        hist_pass(nv1, 13, 255, m1)
        c2, a2 = scan_pass(256, k_rem)
        k_rem = k_rem - a2
        m2 = compact_pass(nv1, 13, 255, c2, m1)
        nv2 = (m2 + 15) // 16
        hist_pass(nv2, 5, 255, m2)
        c3, a3 = scan_pass(256, k_rem)
        k_rem = k_rem - a3
        m3 = compact_pass(nv2, 5, 255, c3, m2)
        nv3 = (m3 + 15) // 16
        hist_pass(nv3, 0, 31, m3)
        c4, a4 = scan_pass(32, k_rem)
        n_above = a1 + a2 + a3 + a4
        t_bits = (c1 << 21) | (c2 << 13) | (c3 << 5) | c4
        out_acc = jnp.where(lane == r, t_bits, out_acc)
        out_acc = jnp.where(lane == R_PER + r, n_above, out_acc)
        return out_acc

    out_acc = jnp.zeros((16,), jnp.int32)
    for r in range(R_PER):
        out_acc = do_row(r, out_acc)
    out_v[...] = out_acc
    pltpu.sync_copy(out_v, out_hbm.at[wid])


_TC_BLK = 2048


def _tc_body(x_ref, t_ref, a_ref, out_ref):
    step = pl.program_id(0)
    x = x_ref[...]
    t = t_ref[...]
    s = jnp.sum(jnp.where(x > t, jnp.log(x), 0.0),
                axis=(0, 1), keepdims=True)

    @pl.when(step == 0)
    def _():
        nsel = jnp.float32(KSEL) - a_ref[...].astype(jnp.float32)
        out_ref[...] = jnp.sum(nsel * jnp.log(t), axis=(0, 1), keepdims=True)

    out_ref[...] += s


def _tc_logsum(x, t, a):
    return pl.pallas_call(
        _tc_body,
        grid=(C // _TC_BLK,),
        in_specs=[
            pl.BlockSpec((R, _TC_BLK), lambda i: (0, i)),
            pl.BlockSpec((R, 1), lambda i: (0, 0)),
            pl.BlockSpec((R, 1), lambda i: (0, 0)),
        ],
        out_specs=pl.BlockSpec((1, 1), lambda i: (0, 0)),
        out_shape=jax.ShapeDtypeStruct((1, 1), jnp.float32),
    )(x, t, a)


def kernel(inputs, k):
    sel = _get_sc_select()(inputs)
    t_bits = sel[:, :R_PER].reshape(R, 1)
    n_above = sel[:, R_PER:2 * R_PER].reshape(R, 1)
    t = lax.bitcast_convert_type(t_bits, jnp.float32)
    total = _tc_logsum(inputs, t, n_above)[0, 0]
    return -total / (jnp.float32(R) * jnp.asarray(k, jnp.float32))
